# baseline (speedup 1.0000x reference)
import functools

import jax
import jax.numpy as jnp
from jax import lax
from jax.experimental import pallas as pl
from jax.experimental.pallas import tpu as pltpu
from jax.experimental.pallas import tpu_sc as plsc

E = 64
K = 2
H = 1024
FF = 2048
BT = 2048
N = BT * K
TM = 128
NT = N // TM + E
NPAD = NT * TM
FFB = 2048
NFF = FF // FFB
NC, NS = 2, 16
NW = NC * NS

f32 = jnp.float32
i32 = jnp.int32

RB = 256
RB2 = 256
RB3 = 256


def _router_body(x_ref, wr_ref, w_ref, id_ref):
    xt = x_ref[...]
    logits = lax.dot_general(xt, wr_ref[...], (((1,), (1,)), ((), ())),
                             preferred_element_type=f32)
    ids = lax.broadcasted_iota(i32, (RB, E), 1)
    m1 = jnp.max(logits, axis=1, keepdims=True)
    id1 = jnp.min(jnp.where(logits == m1, ids, E), axis=1, keepdims=True)
    masked = jnp.where(ids == id1, -jnp.inf, logits)
    m2 = jnp.max(masked, axis=1, keepdims=True)
    id2 = jnp.min(jnp.where(masked == m2, ids, E), axis=1, keepdims=True)
    e2 = jnp.exp(m2 - m1)
    w1 = 1.0 / (1.0 + e2)
    w2 = e2 * w1
    two = lax.broadcasted_iota(i32, (RB, K), 1)
    w_ref[...] = jnp.where(two == 0, w1, w2)
    id_ref[...] = jnp.where(two == 0, id1, id2)


def _router(x, W_router):
    return pl.pallas_call(
        _router_body,
        grid=(BT // RB,),
        in_specs=[
            pl.BlockSpec((RB, H), lambda i: (i, 0)),
            pl.BlockSpec((E, H), lambda i: (0, 0)),
        ],
        out_specs=[
            pl.BlockSpec((RB, K), lambda i: (i, 0)),
            pl.BlockSpec((RB, K), lambda i: (i, 0)),
        ],
        out_shape=[
            jax.ShapeDtypeStruct((BT, K), f32),
            jax.ShapeDtypeStruct((BT, K), i32),
        ],
    )(x, W_router)


def _dispatch_body(idsc_ref, idsr_ref, pos_ref, te_ref, used_ref, padt_s):
    i = pl.program_id(0)

    @pl.when(i == 0)
    def _():
        idsr = idsr_ref[...]
        erow = lax.broadcasted_iota(i32, (E, N), 0)
        onehot = jnp.where(idsr == erow, 1.0, 0.0)
        hist = jnp.sum(onehot, axis=1, keepdims=True)
        padt = jnp.floor((hist + (TM - 1)) / TM)
        er = lax.broadcasted_iota(i32, (E, E), 0)
        ec = lax.broadcasted_iota(i32, (E, E), 1)
        lower = jnp.where(ec < er, 1.0, 0.0)
        cum_excl = lax.dot_general(lower, padt, (((1,), (0,)), ((), ())),
                                   preferred_element_type=f32)
        padt_s[...] = padt
        used_ref[...] = jnp.sum(padt, axis=0, keepdims=True).astype(i32)
        trow = lax.broadcasted_iota(i32, (NT, E), 0).astype(f32)
        cum_row = lax.dot_general(
            jnp.full((NT, 1), 1.0, f32), cum_excl,
            (((1,), (1,)), ((), ())), preferred_element_type=f32)
        cmp = jnp.where(cum_row <= trow, 1.0, 0.0)
        te_ref[...] = (jnp.sum(cmp, axis=1, keepdims=True) - 1.0).astype(i32)

    ei = idsc_ref[...]
    ej = idsr_ref[...]
    lj = lax.broadcasted_iota(i32, (RB2, N), 1)
    li = i * RB2 + lax.broadcasted_iota(i32, (RB2, N), 0)
    before = jnp.where((ej == ei) & (lj < li), 1.0, 0.0)
    rank = jnp.sum(before, axis=1, keepdims=True)
    ecols = lax.broadcasted_iota(i32, (RB2, E), 1)
    lt = jnp.where(ecols < ei, 1.0, 0.0)
    base = lax.dot_general(lt, padt_s[...], (((1,), (0,)), ((), ())),
                           preferred_element_type=f32)
    pos_ref[...] = (TM * base + rank).astype(i32)


def _dispatch(ids_col, ids_row):
    return pl.pallas_call(
        _dispatch_body,
        grid=(N // RB2,),
        in_specs=[
            pl.BlockSpec((RB2, 1), lambda i: (i, 0)),
            pl.BlockSpec((1, N), lambda i: (0, 0)),
        ],
        out_specs=[
            pl.BlockSpec((RB2, 1), lambda i: (i, 0)),
            pl.BlockSpec((NT, 1), lambda i: (0, 0)),
            pl.BlockSpec((1, 1), lambda i: (0, 0)),
        ],
        out_shape=[
            jax.ShapeDtypeStruct((N, 1), i32),
            jax.ShapeDtypeStruct((NT, 1), i32),
            jax.ShapeDtypeStruct((1, 1), i32),
        ],
        scratch_shapes=[pltpu.VMEM((E, 1), f32)],
    )(ids_col, ids_row)


TPW = BT // NW


def _scatterx_body(pe_hbm, po_hbm, x_hbm, xs_hbm, pe_v, po_v, xrows_v,
                   sem0, sem1):
    wid = lax.axis_index("s") * NC + lax.axis_index("c")
    tbase = wid * TPW
    pltpu.sync_copy(pe_hbm.at[pl.ds(tbase, TPW)], pe_v)
    pltpu.sync_copy(po_hbm.at[pl.ds(tbase, TPW)], po_v)
    pltpu.sync_copy(x_hbm.at[pl.ds(tbase, TPW)], xrows_v)
    c0 = pltpu.async_copy(xrows_v, xs_hbm.at[pe_v], sem0)
    c1 = pltpu.async_copy(xrows_v, xs_hbm.at[po_v], sem1)
    c0.wait()
    c1.wait()


def _scatterx(pos_even, pos_odd, x):
    mesh = plsc.VectorSubcoreMesh(core_axis_name="c", subcore_axis_name="s")
    return pl.kernel(
        _scatterx_body,
        out_type=jax.ShapeDtypeStruct((NPAD, H), f32),
        mesh=mesh,
        scratch_types=[
            pltpu.VMEM((TPW,), i32),
            pltpu.VMEM((TPW,), i32),
            pltpu.VMEM((TPW, H), f32),
            pltpu.SemaphoreType.DMA,
            pltpu.SemaphoreType.DMA,
        ],
    )(pos_even, pos_odd, x)


def _mlp_body(te_ref, used_ref, xs_ref, gu_ref, wd_ref, ys_ref):
    s = pl.program_id(0)
    f = pl.program_id(1)

    @pl.when(s < used_ref[0])
    def _():
        xt = xs_ref[...]
        wg = gu_ref[0, 0]
        wu = gu_ref[0, 1]
        g = lax.dot_general(xt, wg, (((1,), (1,)), ((), ())),
                            preferred_element_type=f32)
        u = lax.dot_general(xt, wu, (((1,), (1,)), ((), ())),
                            preferred_element_type=f32)
        hid = g * jax.nn.sigmoid(g) * u
        wd = wd_ref[0]
        part = lax.dot_general(hid, wd, (((1,), (1,)), ((), ())),
                               preferred_element_type=f32)

        @pl.when(f == 0)
        def _():
            ys_ref[...] = part

        @pl.when(f > 0)
        def _():
            ys_ref[...] += part


def _mlp(te, used, xs, gu4, W_down):
    grid_spec = pltpu.PrefetchScalarGridSpec(
        num_scalar_prefetch=2,
        grid=(NT, NFF),
        in_specs=[
            pl.BlockSpec((TM, H), lambda s, f, te, u: (s, 0)),
            pl.BlockSpec((1, 2, FFB, H), lambda s, f, te, u: (te[s], 0, f, 0)),
            pl.BlockSpec((1, H, FFB), lambda s, f, te, u: (te[s], 0, f)),
        ],
        out_specs=pl.BlockSpec((TM, H), lambda s, f, te, u: (s, 0)),
    )
    return pl.pallas_call(
        _mlp_body,
        grid_spec=grid_spec,
        out_shape=jax.ShapeDtypeStruct((NPAD, H), f32),
    )(te, used, xs, gu4, W_down)


CC = 16


def _combine_body(pos_hbm, w_hbm, ys_hbm, y_hbm, pos_v, w_v, rows_v0,
                  rows_v1, out_v, sem0, sem1):
    wid = lax.axis_index("s") * NC + lax.axis_index("c")
    sbase = wid * TPW * K
    pltpu.sync_copy(pos_hbm.at[pl.ds(sbase, TPW * K)], pos_v)
    pltpu.sync_copy(w_hbm.at[pl.ds(sbase, TPW * K)], w_v)
    nch = TPW // CC
    rows = (rows_v0, rows_v1)
    sems = (sem0, sem1)
    copies = [None, None]
    copies[0] = pltpu.async_copy(
        ys_hbm.at[pos_v.at[pl.ds(0, CC * K)]], rows_v0, sem0)
    for c in range(nch):
        copies[c % 2].wait()
        if c + 1 < nch:
            copies[(c + 1) % 2] = pltpu.async_copy(
                ys_hbm.at[pos_v.at[pl.ds((c + 1) * CC * K, CC * K)]],
                rows[(c + 1) % 2], sems[(c + 1) % 2])
        rows_v = rows[c % 2]

        def tok_body(t, _):
            base = c * CC * K + 2 * t
            w0 = w_v[base]
            w1 = w_v[base + 1]

            def col_body(j, _):
                a = rows_v[2 * t, pl.ds(j * 16, 16)]
                b = rows_v[2 * t + 1, pl.ds(j * 16, 16)]
                out_v[t, pl.ds(j * 16, 16)] = w0 * a + w1 * b
                return 0

            return lax.fori_loop(0, H // 16, col_body, 0)

        lax.fori_loop(0, CC, tok_body, 0)
        pltpu.sync_copy(out_v, y_hbm.at[pl.ds(wid * TPW + c * CC, CC)])


def _combine(pos, wflat, ys):
    mesh = plsc.VectorSubcoreMesh(core_axis_name="c", subcore_axis_name="s")
    return pl.kernel(
        _combine_body,
        out_type=jax.ShapeDtypeStruct((BT, H), f32),
        mesh=mesh,
        scratch_types=[
            pltpu.VMEM((TPW * K,), i32),
            pltpu.VMEM((TPW * K, 16), f32),
            pltpu.VMEM((CC * K, H), f32),
            pltpu.VMEM((CC * K, H), f32),
            pltpu.VMEM((CC, H), f32),
            pltpu.SemaphoreType.DMA,
            pltpu.SemaphoreType.DMA,
        ],
    )(pos, wflat, ys)


def kernel(x, W_router, W_gate_up, W_down):
    topw, topids = _router(x, W_router)
    ids_col = topids.reshape(N, 1)
    ids_row = topids.reshape(1, N)
    pos, te2, used2 = _dispatch(ids_col, ids_row)
    pos2 = pos.reshape(BT, K)
    xs = _scatterx(pos2[:, 0], pos2[:, 1], x)
    gu4 = W_gate_up.reshape(E, 2, FF, H)
    ys = _mlp(te2.reshape(NT), used2.reshape(1), xs, gu4, W_down)
    w_exp = jnp.broadcast_to(topw.reshape(N, 1), (N, 16))
    y = _combine(pos.reshape(N), w_exp, ys)
    return y


# speedup vs baseline: 1.0069x; 1.0069x over previous
import functools

import jax
import jax.numpy as jnp
from jax import lax
from jax.experimental import pallas as pl
from jax.experimental.pallas import tpu as pltpu
from jax.experimental.pallas import tpu_sc as plsc

E = 64
K = 2
H = 1024
FF = 2048
BT = 2048
N = BT * K
TM = 128
NT = N // TM + E
NPAD = NT * TM
FFB = 2048
NFF = FF // FFB
NC, NS = 2, 16
NW = NC * NS

f32 = jnp.float32
i32 = jnp.int32

RB = 256
RB2 = 256
RB3 = 256


def _router_body(x_ref, wr_ref, w_ref, id_ref):
    xt = x_ref[...]
    logits = lax.dot_general(xt, wr_ref[...], (((1,), (1,)), ((), ())),
                             preferred_element_type=f32)
    ids = lax.broadcasted_iota(i32, (RB, E), 1)
    m1 = jnp.max(logits, axis=1, keepdims=True)
    id1 = jnp.min(jnp.where(logits == m1, ids, E), axis=1, keepdims=True)
    masked = jnp.where(ids == id1, -jnp.inf, logits)
    m2 = jnp.max(masked, axis=1, keepdims=True)
    id2 = jnp.min(jnp.where(masked == m2, ids, E), axis=1, keepdims=True)
    e2 = jnp.exp(m2 - m1)
    w1 = 1.0 / (1.0 + e2)
    w2 = e2 * w1
    two = lax.broadcasted_iota(i32, (RB, K), 1)
    w_ref[...] = jnp.where(two == 0, w1, w2)
    id_ref[...] = jnp.where(two == 0, id1, id2)


def _router(x, W_router):
    return pl.pallas_call(
        _router_body,
        grid=(BT // RB,),
        in_specs=[
            pl.BlockSpec((RB, H), lambda i: (i, 0)),
            pl.BlockSpec((E, H), lambda i: (0, 0)),
        ],
        out_specs=[
            pl.BlockSpec((RB, K), lambda i: (i, 0)),
            pl.BlockSpec((RB, K), lambda i: (i, 0)),
        ],
        out_shape=[
            jax.ShapeDtypeStruct((BT, K), f32),
            jax.ShapeDtypeStruct((BT, K), i32),
        ],
    )(x, W_router)


def _dispatch_body(idsc_ref, idsr_ref, pos_ref, te_ref, used_ref, padt_s):
    i = pl.program_id(0)

    @pl.when(i == 0)
    def _():
        idsr = idsr_ref[...]
        erow = lax.broadcasted_iota(i32, (E, N), 0)
        onehot = jnp.where(idsr == erow, 1.0, 0.0)
        hist = jnp.sum(onehot, axis=1, keepdims=True)
        padt = jnp.floor((hist + (TM - 1)) / TM)
        er = lax.broadcasted_iota(i32, (E, E), 0)
        ec = lax.broadcasted_iota(i32, (E, E), 1)
        lower = jnp.where(ec < er, 1.0, 0.0)
        cum_excl = lax.dot_general(lower, padt, (((1,), (0,)), ((), ())),
                                   preferred_element_type=f32)
        padt_s[...] = padt
        used_ref[...] = jnp.sum(padt, axis=0, keepdims=True).astype(i32)
        trow = lax.broadcasted_iota(i32, (NT, E), 0).astype(f32)
        cum_row = lax.dot_general(
            jnp.full((NT, 1), 1.0, f32), cum_excl,
            (((1,), (1,)), ((), ())), preferred_element_type=f32)
        cmp = jnp.where(cum_row <= trow, 1.0, 0.0)
        te_ref[...] = (jnp.sum(cmp, axis=1, keepdims=True) - 1.0).astype(i32)

    ei = idsc_ref[...]
    ej = idsr_ref[...]
    lj = lax.broadcasted_iota(i32, (RB2, N), 1)
    li = i * RB2 + lax.broadcasted_iota(i32, (RB2, N), 0)
    before = jnp.where((ej == ei) & (lj < li), 1.0, 0.0)
    rank = jnp.sum(before, axis=1, keepdims=True)
    ecols = lax.broadcasted_iota(i32, (RB2, E), 1)
    lt = jnp.where(ecols < ei, 1.0, 0.0)
    base = lax.dot_general(lt, padt_s[...], (((1,), (0,)), ((), ())),
                           preferred_element_type=f32)
    pos_ref[...] = (TM * base + rank).astype(i32)


def _dispatch(ids_col, ids_row):
    return pl.pallas_call(
        _dispatch_body,
        grid=(N // RB2,),
        in_specs=[
            pl.BlockSpec((RB2, 1), lambda i: (i, 0)),
            pl.BlockSpec((1, N), lambda i: (0, 0)),
        ],
        out_specs=[
            pl.BlockSpec((RB2, 1), lambda i: (i, 0)),
            pl.BlockSpec((NT, 1), lambda i: (0, 0)),
            pl.BlockSpec((1, 1), lambda i: (0, 0)),
        ],
        out_shape=[
            jax.ShapeDtypeStruct((N, 1), i32),
            jax.ShapeDtypeStruct((NT, 1), i32),
            jax.ShapeDtypeStruct((1, 1), i32),
        ],
        scratch_shapes=[pltpu.VMEM((E, 1), f32)],
    )(ids_col, ids_row)


TPW = BT // NW


def _scatterx_body(pe_hbm, po_hbm, x_hbm, xs_hbm, pe_v, po_v, xrows_v,
                   sem0, sem1, sem2):
    wid = lax.axis_index("s") * NC + lax.axis_index("c")
    tbase = wid * TPW
    ci0 = pltpu.async_copy(pe_hbm.at[pl.ds(tbase, TPW)], pe_v, sem0)
    ci1 = pltpu.async_copy(po_hbm.at[pl.ds(tbase, TPW)], po_v, sem1)
    ci2 = pltpu.async_copy(x_hbm.at[pl.ds(tbase, TPW)], xrows_v, sem2)
    ci0.wait()
    ci1.wait()
    ci2.wait()
    c0 = pltpu.async_copy(xrows_v, xs_hbm.at[pe_v], sem0)
    c1 = pltpu.async_copy(xrows_v, xs_hbm.at[po_v], sem1)
    c0.wait()
    c1.wait()


def _scatterx(pos_even, pos_odd, x):
    mesh = plsc.VectorSubcoreMesh(core_axis_name="c", subcore_axis_name="s")
    return pl.kernel(
        _scatterx_body,
        out_type=jax.ShapeDtypeStruct((NPAD, H), f32),
        mesh=mesh,
        scratch_types=[
            pltpu.VMEM((TPW,), i32),
            pltpu.VMEM((TPW,), i32),
            pltpu.VMEM((TPW, H), f32),
            pltpu.SemaphoreType.DMA,
            pltpu.SemaphoreType.DMA,
            pltpu.SemaphoreType.DMA,
        ],
    )(pos_even, pos_odd, x)


def _mlp_body(te_ref, used_ref, xs_ref, gu_ref, wd_ref, ys_ref):
    s = pl.program_id(0)
    f = pl.program_id(1)

    @pl.when(s < used_ref[0])
    def _():
        xt = xs_ref[...]
        wg = gu_ref[0, 0]
        wu = gu_ref[0, 1]
        g = lax.dot_general(xt, wg, (((1,), (1,)), ((), ())),
                            preferred_element_type=f32)
        u = lax.dot_general(xt, wu, (((1,), (1,)), ((), ())),
                            preferred_element_type=f32)
        hid = g * jax.nn.sigmoid(g) * u
        wd = wd_ref[0]
        part = lax.dot_general(hid, wd, (((1,), (1,)), ((), ())),
                               preferred_element_type=f32)

        @pl.when(f == 0)
        def _():
            ys_ref[...] = part

        @pl.when(f > 0)
        def _():
            ys_ref[...] += part


def _mlp(te, used, xs, gu4, W_down):
    grid_spec = pltpu.PrefetchScalarGridSpec(
        num_scalar_prefetch=2,
        grid=(NT, NFF),
        in_specs=[
            pl.BlockSpec((TM, H), lambda s, f, te, u: (s, 0)),
            pl.BlockSpec((1, 2, FFB, H), lambda s, f, te, u: (te[s], 0, f, 0)),
            pl.BlockSpec((1, H, FFB), lambda s, f, te, u: (te[s], 0, f)),
        ],
        out_specs=pl.BlockSpec((TM, H), lambda s, f, te, u: (s, 0)),
    )
    return pl.pallas_call(
        _mlp_body,
        grid_spec=grid_spec,
        out_shape=jax.ShapeDtypeStruct((NPAD, H), f32),
    )(te, used, xs, gu4, W_down)


CC = 16


def _combine_body(pos_hbm, w_hbm, ys_hbm, y_hbm, pos_v, w_v, rows_v0,
                  rows_v1, out_v, sem0, sem1):
    wid = lax.axis_index("s") * NC + lax.axis_index("c")
    sbase = wid * TPW * K
    pltpu.sync_copy(pos_hbm.at[pl.ds(sbase, TPW * K)], pos_v)
    pltpu.sync_copy(w_hbm.at[pl.ds(sbase, TPW * K)], w_v)
    nch = TPW // CC
    rows = (rows_v0, rows_v1)
    sems = (sem0, sem1)
    copies = [None, None]
    copies[0] = pltpu.async_copy(
        ys_hbm.at[pos_v.at[pl.ds(0, CC * K)]], rows_v0, sem0)
    for c in range(nch):
        copies[c % 2].wait()
        if c + 1 < nch:
            copies[(c + 1) % 2] = pltpu.async_copy(
                ys_hbm.at[pos_v.at[pl.ds((c + 1) * CC * K, CC * K)]],
                rows[(c + 1) % 2], sems[(c + 1) % 2])
        rows_v = rows[c % 2]

        def tok_body(t, _):
            base = c * CC * K + 2 * t
            w0 = w_v[base]
            w1 = w_v[base + 1]

            def col_body(j, _):
                for u in range(4):
                    col = j * 64 + u * 16
                    a = rows_v[2 * t, pl.ds(col, 16)]
                    b = rows_v[2 * t + 1, pl.ds(col, 16)]
                    out_v[t, pl.ds(col, 16)] = w0 * a + w1 * b
                return 0

            return lax.fori_loop(0, H // 64, col_body, 0)

        lax.fori_loop(0, CC, tok_body, 0)
        pltpu.sync_copy(out_v, y_hbm.at[pl.ds(wid * TPW + c * CC, CC)])


def _combine(pos, wflat, ys):
    mesh = plsc.VectorSubcoreMesh(core_axis_name="c", subcore_axis_name="s")
    return pl.kernel(
        _combine_body,
        out_type=jax.ShapeDtypeStruct((BT, H), f32),
        mesh=mesh,
        scratch_types=[
            pltpu.VMEM((TPW * K,), i32),
            pltpu.VMEM((TPW * K, 16), f32),
            pltpu.VMEM((CC * K, H), f32),
            pltpu.VMEM((CC * K, H), f32),
            pltpu.VMEM((CC, H), f32),
            pltpu.SemaphoreType.DMA,
            pltpu.SemaphoreType.DMA,
        ],
    )(pos, wflat, ys)


def kernel(x, W_router, W_gate_up, W_down):
    topw, topids = _router(x, W_router)
    ids_col = topids.reshape(N, 1)
    ids_row = topids.reshape(1, N)
    pos, te2, used2 = _dispatch(ids_col, ids_row)
    pos2 = pos.reshape(BT, K)
    xs = _scatterx(pos2[:, 0], pos2[:, 1], x)
    gu4 = W_gate_up.reshape(E, 2, FF, H)
    ys = _mlp(te2.reshape(NT), used2.reshape(1), xs, gu4, W_down)
    w_exp = jnp.broadcast_to(topw.reshape(N, 1), (N, 16))
    y = _combine(pos.reshape(N), w_exp, ys)
    return y


# speedup vs baseline: 1.0201x; 1.0130x over previous
import jax
import jax.numpy as jnp
from jax import lax
from jax.experimental import pallas as pl
from jax.experimental.pallas import tpu as pltpu
from jax.experimental.pallas import tpu_sc as plsc

E = 64
K = 2
H = 1024
FF = 2048
BT = 2048
N = BT * K
TM = 128
NT = N // TM + E
NPAD = NT * TM
FFB = 2048
NFF = FF // FFB
NC, NS = 2, 16
NW = NC * NS

f32 = jnp.float32
i32 = jnp.int32

RB = 256
RB2 = 256


def _router_body(x_ref, wr_ref, w_ref, id_ref):
    xt = x_ref[...]
    logits = lax.dot_general(xt, wr_ref[...], (((1,), (1,)), ((), ())),
                             preferred_element_type=f32)
    ids = lax.broadcasted_iota(i32, (RB, E), 1)
    m1 = jnp.max(logits, axis=1, keepdims=True)
    id1 = jnp.min(jnp.where(logits == m1, ids, E), axis=1, keepdims=True)
    masked = jnp.where(ids == id1, -jnp.inf, logits)
    m2 = jnp.max(masked, axis=1, keepdims=True)
    id2 = jnp.min(jnp.where(masked == m2, ids, E), axis=1, keepdims=True)
    e2 = jnp.exp(m2 - m1)
    w1 = 1.0 / (1.0 + e2)
    w2 = e2 * w1
    two = lax.broadcasted_iota(i32, (RB, K), 1)
    w_ref[...] = jnp.where(two == 0, w1, w2)
    id_ref[...] = jnp.where(two == 0, id1, id2)


def _router(x, W_router):
    return pl.pallas_call(
        _router_body,
        grid=(BT // RB,),
        in_specs=[
            pl.BlockSpec((RB, H), lambda i: (i, 0)),
            pl.BlockSpec((E, H), lambda i: (0, 0)),
        ],
        out_specs=[
            pl.BlockSpec((RB, K), lambda i: (i, 0)),
            pl.BlockSpec((RB, K), lambda i: (i, 0)),
        ],
        out_shape=[
            jax.ShapeDtypeStruct((BT, K), f32),
            jax.ShapeDtypeStruct((BT, K), i32),
        ],
    )(x, W_router)


def _dispatch_body(idsc_ref, idsr_ref, pos_ref, te_ref, used_ref, padt_s):
    i = pl.program_id(0)

    @pl.when(i == 0)
    def _():
        idsr = idsr_ref[...]
        erow = lax.broadcasted_iota(i32, (E, N), 0)
        onehot = jnp.where(idsr == erow, 1.0, 0.0)
        hist = jnp.sum(onehot, axis=1, keepdims=True)
        padt = jnp.floor((hist + (TM - 1)) / TM)
        er = lax.broadcasted_iota(i32, (E, E), 0)
        ec = lax.broadcasted_iota(i32, (E, E), 1)
        lower = jnp.where(ec < er, 1.0, 0.0)
        cum_excl = lax.dot_general(lower, padt, (((1,), (0,)), ((), ())),
                                   preferred_element_type=f32)
        padt_s[...] = padt
        used_ref[...] = jnp.sum(padt, axis=0, keepdims=True).astype(i32)
        trow = lax.broadcasted_iota(i32, (NT, E), 0).astype(f32)
        cum_row = lax.dot_general(
            jnp.full((NT, 1), 1.0, f32), cum_excl,
            (((1,), (1,)), ((), ())), preferred_element_type=f32)
        cmp = jnp.where(cum_row <= trow, 1.0, 0.0)
        te_ref[...] = (jnp.sum(cmp, axis=1, keepdims=True) - 1.0).astype(i32)

    ei = idsc_ref[...]
    ej = idsr_ref[...]
    lj = lax.broadcasted_iota(i32, (RB2, N), 1)
    li = i * RB2 + lax.broadcasted_iota(i32, (RB2, N), 0)
    before = jnp.where((ej == ei) & (lj < li), 1.0, 0.0)
    rank = jnp.sum(before, axis=1, keepdims=True)
    ecols = lax.broadcasted_iota(i32, (RB2, E), 1)
    lt = jnp.where(ecols < ei, 1.0, 0.0)
    base = lax.dot_general(lt, padt_s[...], (((1,), (0,)), ((), ())),
                           preferred_element_type=f32)
    pos_ref[...] = (TM * base + rank).astype(i32)


def _dispatch(ids_col, ids_row):
    return pl.pallas_call(
        _dispatch_body,
        grid=(N // RB2,),
        in_specs=[
            pl.BlockSpec((RB2, 1), lambda i: (i, 0)),
            pl.BlockSpec((1, N), lambda i: (0, 0)),
        ],
        out_specs=[
            pl.BlockSpec((RB2, 1), lambda i: (i, 0)),
            pl.BlockSpec((NT, 1), lambda i: (0, 0)),
            pl.BlockSpec((1, 1), lambda i: (0, 0)),
        ],
        out_shape=[
            jax.ShapeDtypeStruct((N, 1), i32),
            jax.ShapeDtypeStruct((NT, 1), i32),
            jax.ShapeDtypeStruct((1, 1), i32),
        ],
        scratch_shapes=[pltpu.VMEM((E, 1), f32)],
    )(ids_col, ids_row)


TPW = BT // NW


def _scatterx_body(pe_hbm, po_hbm, x_hbm, xs_hbm, pe_v, po_v, xrows_v,
                   sem0, sem1, sem2):
    wid = lax.axis_index("s") * NC + lax.axis_index("c")
    tbase = wid * TPW
    pltpu.sync_copy(pe_hbm.at[pl.ds(tbase, TPW)], pe_v)
    pltpu.sync_copy(po_hbm.at[pl.ds(tbase, TPW)], po_v)
    pltpu.sync_copy(x_hbm.at[pl.ds(tbase, TPW)], xrows_v)
    c0 = pltpu.async_copy(xrows_v, xs_hbm.at[pe_v], sem0)
    c1 = pltpu.async_copy(xrows_v, xs_hbm.at[po_v], sem1)
    c0.wait()
    c1.wait()


def _scatterx(pos_even, pos_odd, x):
    mesh = plsc.VectorSubcoreMesh(core_axis_name="c", subcore_axis_name="s")
    return pl.kernel(
        _scatterx_body,
        out_type=jax.ShapeDtypeStruct((NPAD, H), f32),
        mesh=mesh,
        scratch_types=[
            pltpu.VMEM((TPW,), i32),
            pltpu.VMEM((TPW,), i32),
            pltpu.VMEM((TPW, H), f32),
            pltpu.SemaphoreType.DMA,
            pltpu.SemaphoreType.DMA,
            pltpu.SemaphoreType.DMA,
        ],
    )(pos_even, pos_odd, x)


def _mlp_body(te_ref, used_ref, xs_ref, gu_ref, wd_ref, ys_ref):
    s = pl.program_id(0)
    f = pl.program_id(1)

    @pl.when(s < used_ref[0])
    def _():
        xt = xs_ref[...]
        wg = gu_ref[0, 0]
        wu = gu_ref[0, 1]
        g = lax.dot_general(xt, wg, (((1,), (1,)), ((), ())),
                            preferred_element_type=f32)
        u = lax.dot_general(xt, wu, (((1,), (1,)), ((), ())),
                            preferred_element_type=f32)
        hid = g * jax.nn.sigmoid(g) * u
        wd = wd_ref[0]
        part = lax.dot_general(hid, wd, (((1,), (1,)), ((), ())),
                               preferred_element_type=f32)

        @pl.when(f == 0)
        def _():
            ys_ref[...] = part

        @pl.when(f > 0)
        def _():
            ys_ref[...] += part


def _mlp(te, used, xs, gu4, W_down):
    grid_spec = pltpu.PrefetchScalarGridSpec(
        num_scalar_prefetch=2,
        grid=(NT, NFF),
        in_specs=[
            pl.BlockSpec((TM, H), lambda s, f, te, u: (s, 0)),
            pl.BlockSpec((1, 2, FFB, H), lambda s, f, te, u: (te[s], 0, f, 0)),
            pl.BlockSpec((1, H, FFB), lambda s, f, te, u: (te[s], 0, f)),
        ],
        out_specs=pl.BlockSpec((TM, H), lambda s, f, te, u: (s, 0)),
    )
    return pl.pallas_call(
        _mlp_body,
        grid_spec=grid_spec,
        out_shape=jax.ShapeDtypeStruct((NPAD, H), f32),
    )(te, used, xs, gu4, W_down)


CC = 16


def _combine_body(pos_hbm, w_hbm, ys_hbm, y_hbm, pos_v, w_v, rows_v0,
                  rows_v1, out_v, sem0, sem1):
    wid = lax.axis_index("s") * NC + lax.axis_index("c")
    sbase = wid * TPW * K
    pltpu.sync_copy(pos_hbm.at[pl.ds(sbase, TPW * K)], pos_v)
    pltpu.sync_copy(w_hbm.at[pl.ds(sbase, TPW * K)], w_v)
    nch = TPW // CC
    rows = (rows_v0, rows_v1)
    sems = (sem0, sem1)
    copies = [None, None]
    copies[0] = pltpu.async_copy(
        ys_hbm.at[pos_v.at[pl.ds(0, CC * K)]], rows_v0, sem0)
    for c in range(nch):
        copies[c % 2].wait()
        if c + 1 < nch:
            copies[(c + 1) % 2] = pltpu.async_copy(
                ys_hbm.at[pos_v.at[pl.ds((c + 1) * CC * K, CC * K)]],
                rows[(c + 1) % 2], sems[(c + 1) % 2])
        rows_v = rows[c % 2]

        def tok_body(t, _):
            base = c * CC * K + 2 * t
            w0 = w_v[base]
            w1 = w_v[base + 1]

            def col_body(j, _):
                for u in range(4):
                    col = j * 64 + u * 16
                    a = rows_v[2 * t, pl.ds(col, 16)]
                    b = rows_v[2 * t + 1, pl.ds(col, 16)]
                    out_v[t, pl.ds(col, 16)] = w0 * a + w1 * b
                return 0

            return lax.fori_loop(0, H // 64, col_body, 0)

        lax.fori_loop(0, CC, tok_body, 0)
        pltpu.sync_copy(out_v, y_hbm.at[pl.ds(wid * TPW + c * CC, CC)])


def _combine(pos, wflat, ys):
    mesh = plsc.VectorSubcoreMesh(core_axis_name="c", subcore_axis_name="s")
    return pl.kernel(
        _combine_body,
        out_type=jax.ShapeDtypeStruct((BT, H), f32),
        mesh=mesh,
        scratch_types=[
            pltpu.VMEM((TPW * K,), i32),
            pltpu.VMEM((TPW * K, 16), f32),
            pltpu.VMEM((CC * K, H), f32),
            pltpu.VMEM((CC * K, H), f32),
            pltpu.VMEM((CC, H), f32),
            pltpu.SemaphoreType.DMA,
            pltpu.SemaphoreType.DMA,
        ],
    )(pos, wflat, ys)


def kernel(x, W_router, W_gate_up, W_down):
    topw, topids = _router(x, W_router)
    ids_col = topids.reshape(N, 1)
    ids_row = topids.reshape(1, N)
    pos, te2, used2 = _dispatch(ids_col, ids_row)
    pos2 = pos.reshape(BT, K)
    xs = _scatterx(pos2[:, 0], pos2[:, 1], x)
    gu4 = W_gate_up.reshape(E, 2, FF, H)
    ys = _mlp(te2.reshape(NT), used2.reshape(1), xs, gu4, W_down)
    w_exp = jnp.broadcast_to(topw.reshape(N, 1), (N, 16))
    y = _combine(pos.reshape(N), w_exp, ys)
    return y


# speedup vs baseline: 1.0205x; 1.0004x over previous
import jax
import jax.numpy as jnp
from jax import lax
from jax.experimental import pallas as pl
from jax.experimental.pallas import tpu as pltpu
from jax.experimental.pallas import tpu_sc as plsc

E = 64
K = 2
H = 1024
FF = 2048
BT = 2048
N = BT * K
TM = 128
NT = N // TM + E
NPAD = NT * TM
FFB = 2048
NFF = FF // FFB
NC, NS = 2, 16
NW = NC * NS

f32 = jnp.float32
i32 = jnp.int32

RB = 256
RB2 = 256


def _router_body(x_ref, wr_ref, w_ref, id_ref):
    xt = x_ref[...]
    logits = lax.dot_general(xt, wr_ref[...], (((1,), (1,)), ((), ())),
                             preferred_element_type=f32)
    ids = lax.broadcasted_iota(i32, (RB, E), 1)
    m1 = jnp.max(logits, axis=1, keepdims=True)
    id1 = jnp.min(jnp.where(logits == m1, ids, E), axis=1, keepdims=True)
    masked = jnp.where(ids == id1, -jnp.inf, logits)
    m2 = jnp.max(masked, axis=1, keepdims=True)
    id2 = jnp.min(jnp.where(masked == m2, ids, E), axis=1, keepdims=True)
    e2 = jnp.exp(m2 - m1)
    w1 = 1.0 / (1.0 + e2)
    w2 = e2 * w1
    two = lax.broadcasted_iota(i32, (RB, K), 1)
    w_ref[...] = jnp.where(two == 0, w1, w2)
    id_ref[...] = jnp.where(two == 0, id1, id2)


def _router(x, W_router):
    return pl.pallas_call(
        _router_body,
        grid=(BT // RB,),
        in_specs=[
            pl.BlockSpec((RB, H), lambda i: (i, 0)),
            pl.BlockSpec((E, H), lambda i: (0, 0)),
        ],
        out_specs=[
            pl.BlockSpec((RB, K), lambda i: (i, 0)),
            pl.BlockSpec((RB, K), lambda i: (i, 0)),
        ],
        out_shape=[
            jax.ShapeDtypeStruct((BT, K), f32),
            jax.ShapeDtypeStruct((BT, K), i32),
        ],
    )(x, W_router)


def _dispatch_body(idsc_ref, idsr_ref, pos_ref, te_ref, used_ref, padt_s):
    i = pl.program_id(0)

    @pl.when(i == 0)
    def _():
        idsr = idsr_ref[...]
        erow = lax.broadcasted_iota(i32, (E, N), 0)
        onehot = jnp.where(idsr == erow, 1.0, 0.0)
        hist = jnp.sum(onehot, axis=1, keepdims=True)
        padt = jnp.floor((hist + (TM - 1)) / TM)
        er = lax.broadcasted_iota(i32, (E, E), 0)
        ec = lax.broadcasted_iota(i32, (E, E), 1)
        lower = jnp.where(ec < er, 1.0, 0.0)
        cum_excl = lax.dot_general(lower, padt, (((1,), (0,)), ((), ())),
                                   preferred_element_type=f32)
        padt_s[...] = padt
        used_ref[...] = jnp.sum(padt, axis=0, keepdims=True).astype(i32)
        trow = lax.broadcasted_iota(i32, (NT, E), 0).astype(f32)
        cum_row = lax.dot_general(
            jnp.full((NT, 1), 1.0, f32), cum_excl,
            (((1,), (1,)), ((), ())), preferred_element_type=f32)
        cmp = jnp.where(cum_row <= trow, 1.0, 0.0)
        te_ref[...] = (jnp.sum(cmp, axis=1, keepdims=True) - 1.0).astype(i32)

    ei = idsc_ref[...]
    ej = idsr_ref[...]
    lj = lax.broadcasted_iota(i32, (RB2, N), 1)
    li = i * RB2 + lax.broadcasted_iota(i32, (RB2, N), 0)
    before = jnp.where((ej == ei) & (lj < li), 1.0, 0.0)
    rank = jnp.sum(before, axis=1, keepdims=True)
    ecols = lax.broadcasted_iota(i32, (RB2, E), 1)
    lt = jnp.where(ecols < ei, 1.0, 0.0)
    base = lax.dot_general(lt, padt_s[...], (((1,), (0,)), ((), ())),
                           preferred_element_type=f32)
    pos_ref[...] = (TM * base + rank).astype(i32)


def _dispatch(ids_col, ids_row):
    return pl.pallas_call(
        _dispatch_body,
        grid=(N // RB2,),
        in_specs=[
            pl.BlockSpec((RB2, 1), lambda i: (i, 0)),
            pl.BlockSpec((1, N), lambda i: (0, 0)),
        ],
        out_specs=[
            pl.BlockSpec((RB2, 1), lambda i: (i, 0)),
            pl.BlockSpec((NT, 1), lambda i: (0, 0)),
            pl.BlockSpec((1, 1), lambda i: (0, 0)),
        ],
        out_shape=[
            jax.ShapeDtypeStruct((N, 1), i32),
            jax.ShapeDtypeStruct((NT, 1), i32),
            jax.ShapeDtypeStruct((1, 1), i32),
        ],
        scratch_shapes=[pltpu.VMEM((E, 1), f32)],
    )(ids_col, ids_row)


TPW = BT // NW


def _scatterx_body(pe_hbm, po_hbm, x_hbm, xs_hbm, pe_v, po_v, xrows_v,
                   sem0, sem1, sem2):
    wid = lax.axis_index("s") * NC + lax.axis_index("c")
    tbase = wid * TPW
    pltpu.sync_copy(pe_hbm.at[pl.ds(tbase, TPW)], pe_v)
    pltpu.sync_copy(po_hbm.at[pl.ds(tbase, TPW)], po_v)
    pltpu.sync_copy(x_hbm.at[pl.ds(tbase, TPW)], xrows_v)
    c0 = pltpu.async_copy(xrows_v, xs_hbm.at[pe_v], sem0)
    c1 = pltpu.async_copy(xrows_v, xs_hbm.at[po_v], sem1)
    c0.wait()
    c1.wait()


def _scatterx(pos_even, pos_odd, x):
    mesh = plsc.VectorSubcoreMesh(core_axis_name="c", subcore_axis_name="s")
    return pl.kernel(
        _scatterx_body,
        out_type=jax.ShapeDtypeStruct((NPAD, H), f32),
        mesh=mesh,
        scratch_types=[
            pltpu.VMEM((TPW,), i32),
            pltpu.VMEM((TPW,), i32),
            pltpu.VMEM((TPW, H), f32),
            pltpu.SemaphoreType.DMA,
            pltpu.SemaphoreType.DMA,
            pltpu.SemaphoreType.DMA,
        ],
    )(pos_even, pos_odd, x)


def _mlp_body(te_ref, used_ref, xs_ref, gu_ref, wd_ref, ys_ref):
    s = pl.program_id(0)

    @pl.when(s < used_ref[0])
    def _():
        xt = xs_ref[...]
        wg = gu_ref[0, 0]
        wu = gu_ref[0, 1]
        g = lax.dot_general(xt, wg, (((1,), (1,)), ((), ())),
                            preferred_element_type=f32)
        u = lax.dot_general(xt, wu, (((1,), (1,)), ((), ())),
                            preferred_element_type=f32)
        hid = g * jax.nn.sigmoid(g) * u
        wd = wd_ref[0]
        ys_ref[...] = lax.dot_general(hid, wd, (((1,), (1,)), ((), ())),
                                      preferred_element_type=f32)


def _mlp(te, used, xs, gu4, W_down):
    grid_spec = pltpu.PrefetchScalarGridSpec(
        num_scalar_prefetch=2,
        grid=(NT,),
        in_specs=[
            pl.BlockSpec((TM, H), lambda s, te, u: (s, 0)),
            pl.BlockSpec((1, 2, FFB, H), lambda s, te, u: (te[s], 0, 0, 0)),
            pl.BlockSpec((1, H, FFB), lambda s, te, u: (te[s], 0, 0)),
        ],
        out_specs=pl.BlockSpec((TM, H), lambda s, te, u: (s, 0)),
    )
    return pl.pallas_call(
        _mlp_body,
        grid_spec=grid_spec,
        out_shape=jax.ShapeDtypeStruct((NPAD, H), f32),
    )(te, used, xs, gu4, W_down)


CC = 16


def _combine_body(pos_hbm, w_hbm, ys_hbm, y_hbm, pos_v, w_v, rows_v0,
                  rows_v1, out_v, sem0, sem1):
    wid = lax.axis_index("s") * NC + lax.axis_index("c")
    sbase = wid * TPW * K
    pltpu.sync_copy(pos_hbm.at[pl.ds(sbase, TPW * K)], pos_v)
    pltpu.sync_copy(w_hbm.at[pl.ds(sbase, TPW * K)], w_v)
    nch = TPW // CC
    rows = (rows_v0, rows_v1)
    sems = (sem0, sem1)
    copies = [None, None]
    copies[0] = pltpu.async_copy(
        ys_hbm.at[pos_v.at[pl.ds(0, CC * K)]], rows_v0, sem0)
    for c in range(nch):
        copies[c % 2].wait()
        if c + 1 < nch:
            copies[(c + 1) % 2] = pltpu.async_copy(
                ys_hbm.at[pos_v.at[pl.ds((c + 1) * CC * K, CC * K)]],
                rows[(c + 1) % 2], sems[(c + 1) % 2])
        rows_v = rows[c % 2]

        def tok_body(t, _):
            base = c * CC * K + 2 * t
            w0 = w_v[base]
            w1 = w_v[base + 1]

            def col_body(j, _):
                for u in range(4):
                    col = j * 64 + u * 16
                    a = rows_v[2 * t, pl.ds(col, 16)]
                    b = rows_v[2 * t + 1, pl.ds(col, 16)]
                    out_v[t, pl.ds(col, 16)] = w0 * a + w1 * b
                return 0

            return lax.fori_loop(0, H // 64, col_body, 0)

        lax.fori_loop(0, CC, tok_body, 0)
        pltpu.sync_copy(out_v, y_hbm.at[pl.ds(wid * TPW + c * CC, CC)])


def _combine(pos, wflat, ys):
    mesh = plsc.VectorSubcoreMesh(core_axis_name="c", subcore_axis_name="s")
    return pl.kernel(
        _combine_body,
        out_type=jax.ShapeDtypeStruct((BT, H), f32),
        mesh=mesh,
        scratch_types=[
            pltpu.VMEM((TPW * K,), i32),
            pltpu.VMEM((TPW * K, 16), f32),
            pltpu.VMEM((CC * K, H), f32),
            pltpu.VMEM((CC * K, H), f32),
            pltpu.VMEM((CC, H), f32),
            pltpu.SemaphoreType.DMA,
            pltpu.SemaphoreType.DMA,
        ],
    )(pos, wflat, ys)


def kernel(x, W_router, W_gate_up, W_down):
    topw, topids = _router(x, W_router)
    ids_col = topids.reshape(N, 1)
    ids_row = topids.reshape(1, N)
    pos, te2, used2 = _dispatch(ids_col, ids_row)
    pos2 = pos.reshape(BT, K)
    xs = _scatterx(pos2[:, 0], pos2[:, 1], x)
    gu4 = W_gate_up.reshape(E, 2, FF, H)
    ys = _mlp(te2.reshape(NT), used2.reshape(1), xs, gu4, W_down)
    w_exp = jnp.broadcast_to(topw.reshape(N, 1), (N, 16))
    y = _combine(pos.reshape(N), w_exp, ys)
    return y
